# baseline (speedup 1.0000x reference)
import functools

import jax
import jax.numpy as jnp
from jax import lax
from jax.experimental import pallas as pl
from jax.experimental.pallas import tpu as pltpu
from jax.experimental.pallas import tpu_sc as plsc

_N = 10000
_D = 128
_H = 128
_G = 64
_RB = 400
_NRB = _N // _RB

_RPT = 320
_NROWS = 32 * _RPT
_HALF = 16 * _RPT
_GARB = 32
_APAD = _HALF + 16 * _GARB
_CAP = 12032
_NCHB = _CAP // 128
_ZR = 16


def _init_body(x_ref, w0_ref, wc_ref, x1_ref, m_ref):
    x1 = jax.nn.sigmoid(
        jnp.dot(x_ref[...], w0_ref[...], preferred_element_type=jnp.float32))
    x1_ref[...] = x1
    m_ref[...] = jnp.dot(x1, wc_ref[...], preferred_element_type=jnp.float32)


_init_call = pl.pallas_call(
    _init_body,
    grid=(_NRB,),
    in_specs=[
        pl.BlockSpec((_RB, _D), lambda i: (i, 0)),
        pl.BlockSpec((_D, _H), lambda i: (0, 0)),
        pl.BlockSpec((_H, _H), lambda i: (0, 0)),
    ],
    out_specs=[
        pl.BlockSpec((_RB, _H), lambda i: (i, 0)),
        pl.BlockSpec((_RB, _H), lambda i: (i, 0)),
    ],
    out_shape=[
        jax.ShapeDtypeStruct((_N, _H), jnp.float32),
        jax.ShapeDtypeStruct((_N, _H), jnp.float32),
    ],
)


def _gru_common(agg_ref, h_ref, wih_ref, whh_ref, bih_ref, bhh_ref):
    agg = agg_ref[...]
    h = h_ref[...]
    gi = jnp.dot(agg, wih_ref[...], preferred_element_type=jnp.float32) + bih_ref[...]
    gh = jnp.dot(h, whh_ref[...], preferred_element_type=jnp.float32) + bhh_ref[...]
    r = jax.nn.sigmoid(gi[:, :_H] + gh[:, :_H])
    z = jax.nn.sigmoid(gi[:, _H:2 * _H] + gh[:, _H:2 * _H])
    n = jnp.tanh(gi[:, 2 * _H:] + r * gh[:, 2 * _H:])
    return (1.0 - z) * n + z * h


def _gru_mid_body(agg_ref, h_ref, wih_ref, whh_ref, bih_ref, bhh_ref, wc_ref,
                  h_out_ref, m_out_ref):
    hn = _gru_common(agg_ref, h_ref, wih_ref, whh_ref, bih_ref, bhh_ref)
    h_out_ref[...] = hn
    m_out_ref[...] = jnp.dot(hn, wc_ref[...], preferred_element_type=jnp.float32)


_gru_mid_call = pl.pallas_call(
    _gru_mid_body,
    grid=(_NRB,),
    in_specs=[
        pl.BlockSpec((_RB, _H), lambda i: (i, 0)),
        pl.BlockSpec((_RB, _H), lambda i: (i, 0)),
        pl.BlockSpec((_H, 3 * _H), lambda i: (0, 0)),
        pl.BlockSpec((_H, 3 * _H), lambda i: (0, 0)),
        pl.BlockSpec((1, 3 * _H), lambda i: (0, 0)),
        pl.BlockSpec((1, 3 * _H), lambda i: (0, 0)),
        pl.BlockSpec((_H, _H), lambda i: (0, 0)),
    ],
    out_specs=[
        pl.BlockSpec((_RB, _H), lambda i: (i, 0)),
        pl.BlockSpec((_RB, _H), lambda i: (i, 0)),
    ],
    out_shape=[
        jax.ShapeDtypeStruct((_N, _H), jnp.float32),
        jax.ShapeDtypeStruct((_N, _H), jnp.float32),
    ],
)


def _gru_head_body(agg_ref, h_ref, wih_ref, whh_ref, bih_ref, bhh_ref,
                   wms_ref, bms_ref, ms_ref):
    hn = _gru_common(agg_ref, h_ref, wih_ref, whh_ref, bih_ref, bhh_ref)
    xo = jnp.maximum(hn, 0.0)
    ms = jnp.dot(xo, wms_ref[...], preferred_element_type=jnp.float32) + bms_ref[...]
    mu = ms[:, 0:1]
    s = ms[:, 1:2]
    sp = jnp.maximum(s, 0.0) + jnp.log1p(jnp.exp(-jnp.abs(s)))
    ms_ref[...] = jnp.concatenate([mu, sp], axis=1)


_gru_head_call = pl.pallas_call(
    _gru_head_body,
    grid=(_NRB,),
    in_specs=[
        pl.BlockSpec((_RB, _H), lambda i: (i, 0)),
        pl.BlockSpec((_RB, _H), lambda i: (i, 0)),
        pl.BlockSpec((_H, 3 * _H), lambda i: (0, 0)),
        pl.BlockSpec((_H, 3 * _H), lambda i: (0, 0)),
        pl.BlockSpec((1, 3 * _H), lambda i: (0, 0)),
        pl.BlockSpec((1, 3 * _H), lambda i: (0, 0)),
        pl.BlockSpec((_H, 2), lambda i: (0, 0)),
        pl.BlockSpec((1, 2), lambda i: (0, 0)),
    ],
    out_specs=pl.BlockSpec((_RB, 2), lambda i: (i, 0)),
    out_shape=jax.ShapeDtypeStruct((_N, 2), jnp.float32),
)


def _sums_body(ms_ref, b_ref, out_ref):
    i = pl.program_id(0)

    @pl.when(i == 0)
    def _():
        out_ref[...] = jnp.zeros_like(out_ref)

    b = b_ref[:, 0]
    onehot = (b[None, :] == lax.broadcasted_iota(jnp.int32, (_G, _RB), 0).astype(jnp.float32))
    out_ref[...] += jnp.dot(onehot.astype(jnp.float32), ms_ref[...],
                            preferred_element_type=jnp.float32,
                            precision=lax.Precision.HIGHEST)


_sums_call = pl.pallas_call(
    _sums_body,
    grid=(_NRB,),
    in_specs=[
        pl.BlockSpec((_RB, 2), lambda i: (i, 0)),
        pl.BlockSpec((_RB, 1), lambda i: (i, 0)),
    ],
    out_specs=pl.BlockSpec((_G, 2), lambda i: (0, 0)),
    out_shape=jax.ShapeDtypeStruct((_G, 2), jnp.float32),
)


def _apply_body(ms_ref, b_ref, sums_ref, out_ref):
    b = b_ref[:, 0]
    onehot = (b[:, None] == lax.broadcasted_iota(jnp.int32, (_RB, _G), 1).astype(jnp.float32))
    gath = jnp.dot(onehot.astype(jnp.float32), sums_ref[...],
                   preferred_element_type=jnp.float32,
                   precision=lax.Precision.HIGHEST)
    mu = ms_ref[:, 0:1]
    sig = ms_ref[:, 1:2]
    out_ref[...] = mu - gath[:, 0:1] * (sig / gath[:, 1:2])


_apply_call = pl.pallas_call(
    _apply_body,
    grid=(_NRB,),
    in_specs=[
        pl.BlockSpec((_RB, 2), lambda i: (i, 0)),
        pl.BlockSpec((_RB, 1), lambda i: (i, 0)),
        pl.BlockSpec((_G, 2), lambda i: (0, 0)),
    ],
    out_specs=pl.BlockSpec((_RB, 1), lambda i: (i, 0)),
    out_shape=jax.ShapeDtypeStruct((_N, 1), jnp.float32),
)


def _seg_body(m_hbm, csrc_hbm, cscat_hbm, same_hbm, out_hbm,
              csrc_v, cscat_v, dst_v, same_v, r0, r1, sv, zbuf, acc,
              s0, s1):
    c = lax.axis_index("c")
    s = lax.axis_index("s")
    w = c * 16 + s
    bufs = (r0, r1)
    sems = (s0, s1)

    zero16 = jnp.zeros((16,), jnp.float32)

    def zrow(k, carry):
        zbuf[k // 8, pl.ds((k % 8) * 16, 16)] = zero16
        return carry

    lax.fori_loop(0, _ZR * 8, zrow, 0)

    def zslice(j, carry):
        off = pl.multiple_of(s * _RPT + j * _ZR, _ZR)
        pltpu.sync_copy(zbuf, acc.at[pl.ds(off, _ZR)])
        return carry

    lax.fori_loop(0, _RPT // _ZR, zslice, 0)
    def zgarb(j, carry):
        goff = pl.multiple_of(_HALF + s * _GARB + j * _ZR, _ZR)
        pltpu.sync_copy(zbuf, acc.at[pl.ds(goff, _ZR)])
        return carry

    lax.fori_loop(0, _GARB // _ZR, zgarb, 0)

    def quad(p, carries):
        base = p * 2
        off = pl.multiple_of(base * 128, 256)
        pltpu.sync_copy(csrc_hbm.at[w, pl.ds(off, 256)], csrc_v)
        pltpu.sync_copy(cscat_hbm.at[w, pl.ds(off, 256)], cscat_v)
        handles = []
        for b in range(2):
            idx = csrc_v.at[pl.ds(b * 128, 128)]
            handles.append(pltpu.async_copy(m_hbm.at[idx], bufs[b], sems[b]))
        for b in range(2):
            t = base + b
            handles[b].wait()
            pltpu.sync_copy(same_hbm.at[w, pl.ds(t * 128, 128)], same_v)
            rv = bufs[b]

            def rrow(j, cs, _rv=rv):
                sb = same_v[j]
                new = []
                for k in range(8):
                    rr = _rv[j, pl.ds(k * 16, 16)]
                    ck = rr + sb * cs[k]
                    sv[j, pl.ds(k * 16, 16)] = ck
                    new.append(ck)
                return tuple(new)

            carries = lax.fori_loop(0, 128, rrow, carries)

            def cp(k2, cc, _b=b):
                dst_v[pl.ds(k2 * 16, 16)] = cscat_v[pl.ds(_b * 128 + k2 * 16, 16)]
                return cc

            lax.fori_loop(0, 8, cp, 0)
            pltpu.sync_copy(sv, acc.at[dst_v], add=True)
        return carries

    zc = tuple(jnp.zeros((16,), jnp.float32) for _ in range(8))
    lax.fori_loop(0, _NCHB // 2, quad, zc)

    out_off = pl.multiple_of(s * _RPT, _RPT)
    pltpu.sync_copy(acc.at[pl.ds(out_off, _RPT)],
                    out_hbm.at[pl.ds(pl.multiple_of(w * _RPT, _RPT), _RPT)])


@functools.cache
def _make_sc_calls():
    mesh = plsc.VectorSubcoreMesh(core_axis_name="c", subcore_axis_name="s")
    seg = pl.kernel(
        _seg_body,
        out_type=jax.ShapeDtypeStruct((_NROWS, _H), jnp.float32),
        mesh=mesh,
        scratch_types=[
            pltpu.VMEM((256,), jnp.int32),
            pltpu.VMEM((256,), jnp.int32),
            pltpu.VMEM((128,), jnp.int32),
            pltpu.VMEM((128, 16), jnp.float32),
            pltpu.VMEM((128, _H), jnp.float32),
            pltpu.VMEM((128, _H), jnp.float32),
            pltpu.VMEM((128, _H), jnp.float32),
            pltpu.VMEM((_ZR, _H), jnp.float32),
            pltpu.VMEM_SHARED((_APAD, _H), jnp.float32),
            pltpu.SemaphoreType.DMA,
            pltpu.SemaphoreType.DMA,
        ],
    )
    return seg


def _edge_segment_sum(m, csrc, cscat, samebc):
    return _make_sc_calls()(m, csrc, cscat, samebc)


def _build_lists(src, dst):
    e = src.shape[0]
    order = jnp.argsort(dst, stable=True)
    srcs = src[order]
    dsts = dst[order]
    same = jnp.concatenate([jnp.zeros((1,), jnp.int32),
                            (dsts[1:] == dsts[:-1]).astype(jnp.int32)])
    last = jnp.concatenate([(dsts[1:] != dsts[:-1]).astype(jnp.int32),
                            jnp.ones((1,), jnp.int32)])
    owner = dsts // _RPT
    start = jnp.searchsorted(owner, jnp.arange(33, dtype=jnp.int32))
    idxmat = start[:32, None] + jnp.arange(_CAP, dtype=jnp.int32)[None, :]
    valid = idxmat < start[1:, None]
    cl = jnp.clip(idxmat, 0, e - 1)
    wvec = jnp.arange(32, dtype=jnp.int32)[:, None]
    jcyc = jnp.arange(_CAP, dtype=jnp.int32)[None, :] % _GARB
    garbl = _HALF + (wvec % 16) * _GARB + jcyc
    comp_src = jnp.where(valid, srcs[cl], 0)
    scat = jnp.where(last[cl] == 1, dsts[cl] - (wvec // 16) * _HALF, garbl)
    comp_scat = jnp.where(valid, scat, garbl)
    samef = jnp.where(valid, same[cl].astype(jnp.float32), 0.0)
    samebc = jnp.broadcast_to(samef[:, :, None], (32, _CAP, 16))
    return comp_src, comp_scat, samebc


def kernel(x, edge_index, batch, num_graphs,
           W0, conv_weight, W_ih, W_hh, b_ih, b_hh, w1, b1, w2, b2):
    src = edge_index[0]
    dst = edge_index[1]

    bih2 = b_ih.reshape(1, 3 * _H)
    bhh2 = b_hh.reshape(1, 3 * _H)
    wms = jnp.concatenate([w1, w2], axis=1)
    bms = jnp.concatenate([b1, b2]).reshape(1, 2)
    batchf = batch.astype(jnp.float32).reshape(_N, 1)

    csrc, cscat, samebc = _build_lists(src, dst)

    x1, m = _init_call(x, W0, conv_weight[0])
    h = x1
    for i in range(2):
        agg = _edge_segment_sum(m, csrc, cscat, samebc)
        h, m = _gru_mid_call(agg, h, W_ih, W_hh, bih2, bhh2,
                             conv_weight[i + 1])
    agg = _edge_segment_sum(m, csrc, cscat, samebc)
    musig = _gru_head_call(agg, h, W_ih, W_hh, bih2, bhh2, wms, bms)
    sums = _sums_call(musig, batchf)
    mu_c = _apply_call(musig, batchf, sums)
    return (mu_c[:, 0], x1, musig[:, 1], musig[:, 0])


# speedup vs baseline: 1.0270x; 1.0270x over previous
import functools

import jax
import jax.numpy as jnp
from jax import lax
from jax.experimental import pallas as pl
from jax.experimental.pallas import tpu as pltpu
from jax.experimental.pallas import tpu_sc as plsc

_N = 10000
_D = 128
_H = 128
_G = 64
_RB = 400
_NRB = _N // _RB

_RPT = 320
_NROWS = 32 * _RPT
_HALF = 16 * _RPT
_GARB = 32
_APAD = _HALF + 16 * _GARB
_CAP = 12032
_NCHB = _CAP // 128
_ZR = 16


def _init_body(x_ref, w0_ref, wc_ref, x1_ref, m_ref):
    x1 = jax.nn.sigmoid(
        jnp.dot(x_ref[...], w0_ref[...], preferred_element_type=jnp.float32))
    x1_ref[...] = x1
    m_ref[...] = jnp.dot(x1, wc_ref[...], preferred_element_type=jnp.float32)


_init_call = pl.pallas_call(
    _init_body,
    grid=(_NRB,),
    in_specs=[
        pl.BlockSpec((_RB, _D), lambda i: (i, 0)),
        pl.BlockSpec((_D, _H), lambda i: (0, 0)),
        pl.BlockSpec((_H, _H), lambda i: (0, 0)),
    ],
    out_specs=[
        pl.BlockSpec((_RB, _H), lambda i: (i, 0)),
        pl.BlockSpec((_RB, _H), lambda i: (i, 0)),
    ],
    out_shape=[
        jax.ShapeDtypeStruct((_N, _H), jnp.float32),
        jax.ShapeDtypeStruct((_N, _H), jnp.float32),
    ],
)


def _gru_common(agg_ref, h_ref, wih_ref, whh_ref, bih_ref, bhh_ref):
    agg = agg_ref[...]
    h = h_ref[...]
    gi = jnp.dot(agg, wih_ref[...], preferred_element_type=jnp.float32) + bih_ref[...]
    gh = jnp.dot(h, whh_ref[...], preferred_element_type=jnp.float32) + bhh_ref[...]
    r = jax.nn.sigmoid(gi[:, :_H] + gh[:, :_H])
    z = jax.nn.sigmoid(gi[:, _H:2 * _H] + gh[:, _H:2 * _H])
    n = jnp.tanh(gi[:, 2 * _H:] + r * gh[:, 2 * _H:])
    return (1.0 - z) * n + z * h


def _gru_mid_body(agg_ref, h_ref, wih_ref, whh_ref, bih_ref, bhh_ref, wc_ref,
                  h_out_ref, m_out_ref):
    hn = _gru_common(agg_ref, h_ref, wih_ref, whh_ref, bih_ref, bhh_ref)
    h_out_ref[...] = hn
    m_out_ref[...] = jnp.dot(hn, wc_ref[...], preferred_element_type=jnp.float32)


_gru_mid_call = pl.pallas_call(
    _gru_mid_body,
    grid=(_NRB,),
    in_specs=[
        pl.BlockSpec((_RB, _H), lambda i: (i, 0)),
        pl.BlockSpec((_RB, _H), lambda i: (i, 0)),
        pl.BlockSpec((_H, 3 * _H), lambda i: (0, 0)),
        pl.BlockSpec((_H, 3 * _H), lambda i: (0, 0)),
        pl.BlockSpec((1, 3 * _H), lambda i: (0, 0)),
        pl.BlockSpec((1, 3 * _H), lambda i: (0, 0)),
        pl.BlockSpec((_H, _H), lambda i: (0, 0)),
    ],
    out_specs=[
        pl.BlockSpec((_RB, _H), lambda i: (i, 0)),
        pl.BlockSpec((_RB, _H), lambda i: (i, 0)),
    ],
    out_shape=[
        jax.ShapeDtypeStruct((_N, _H), jnp.float32),
        jax.ShapeDtypeStruct((_N, _H), jnp.float32),
    ],
)


def _gru_head_body(agg_ref, h_ref, wih_ref, whh_ref, bih_ref, bhh_ref,
                   wms_ref, bms_ref, ms_ref):
    hn = _gru_common(agg_ref, h_ref, wih_ref, whh_ref, bih_ref, bhh_ref)
    xo = jnp.maximum(hn, 0.0)
    ms = jnp.dot(xo, wms_ref[...], preferred_element_type=jnp.float32) + bms_ref[...]
    mu = ms[:, 0:1]
    s = ms[:, 1:2]
    sp = jnp.maximum(s, 0.0) + jnp.log1p(jnp.exp(-jnp.abs(s)))
    ms_ref[...] = jnp.concatenate([mu, sp], axis=1)


_gru_head_call = pl.pallas_call(
    _gru_head_body,
    grid=(_NRB,),
    in_specs=[
        pl.BlockSpec((_RB, _H), lambda i: (i, 0)),
        pl.BlockSpec((_RB, _H), lambda i: (i, 0)),
        pl.BlockSpec((_H, 3 * _H), lambda i: (0, 0)),
        pl.BlockSpec((_H, 3 * _H), lambda i: (0, 0)),
        pl.BlockSpec((1, 3 * _H), lambda i: (0, 0)),
        pl.BlockSpec((1, 3 * _H), lambda i: (0, 0)),
        pl.BlockSpec((_H, 2), lambda i: (0, 0)),
        pl.BlockSpec((1, 2), lambda i: (0, 0)),
    ],
    out_specs=pl.BlockSpec((_RB, 2), lambda i: (i, 0)),
    out_shape=jax.ShapeDtypeStruct((_N, 2), jnp.float32),
)


def _sums_body(ms_ref, b_ref, out_ref):
    i = pl.program_id(0)

    @pl.when(i == 0)
    def _():
        out_ref[...] = jnp.zeros_like(out_ref)

    b = b_ref[:, 0]
    onehot = (b[None, :] == lax.broadcasted_iota(jnp.int32, (_G, _RB), 0).astype(jnp.float32))
    out_ref[...] += jnp.dot(onehot.astype(jnp.float32), ms_ref[...],
                            preferred_element_type=jnp.float32,
                            precision=lax.Precision.HIGHEST)


_sums_call = pl.pallas_call(
    _sums_body,
    grid=(_NRB,),
    in_specs=[
        pl.BlockSpec((_RB, 2), lambda i: (i, 0)),
        pl.BlockSpec((_RB, 1), lambda i: (i, 0)),
    ],
    out_specs=pl.BlockSpec((_G, 2), lambda i: (0, 0)),
    out_shape=jax.ShapeDtypeStruct((_G, 2), jnp.float32),
)


def _apply_body(ms_ref, b_ref, sums_ref, out_ref):
    b = b_ref[:, 0]
    onehot = (b[:, None] == lax.broadcasted_iota(jnp.int32, (_RB, _G), 1).astype(jnp.float32))
    gath = jnp.dot(onehot.astype(jnp.float32), sums_ref[...],
                   preferred_element_type=jnp.float32,
                   precision=lax.Precision.HIGHEST)
    mu = ms_ref[:, 0:1]
    sig = ms_ref[:, 1:2]
    out_ref[...] = mu - gath[:, 0:1] * (sig / gath[:, 1:2])


_apply_call = pl.pallas_call(
    _apply_body,
    grid=(_NRB,),
    in_specs=[
        pl.BlockSpec((_RB, 2), lambda i: (i, 0)),
        pl.BlockSpec((_RB, 1), lambda i: (i, 0)),
        pl.BlockSpec((_G, 2), lambda i: (0, 0)),
    ],
    out_specs=pl.BlockSpec((_RB, 1), lambda i: (i, 0)),
    out_shape=jax.ShapeDtypeStruct((_N, 1), jnp.float32),
)


def _seg_body(m_hbm, csrc_hbm, cscat_hbm, same_hbm, out_hbm,
              csrc_v, cscat_v, dst_v, same_v, r0, r1, sv, zbuf, acc,
              s0, s1):
    c = lax.axis_index("c")
    s = lax.axis_index("s")
    w = c * 16 + s
    bufs = (r0, r1)
    sems = (s0, s1)

    zero16 = jnp.zeros((16,), jnp.float32)

    def zrow(k, carry):
        zbuf[k // 8, pl.ds((k % 8) * 16, 16)] = zero16
        return carry

    lax.fori_loop(0, _ZR * 8, zrow, 0)

    def zslice(j, carry):
        off = pl.multiple_of(s * _RPT + j * _ZR, _ZR)
        pltpu.sync_copy(zbuf, acc.at[pl.ds(off, _ZR)])
        return carry

    lax.fori_loop(0, _RPT // _ZR, zslice, 0)
    def zgarb(j, carry):
        goff = pl.multiple_of(_HALF + s * _GARB + j * _ZR, _ZR)
        pltpu.sync_copy(zbuf, acc.at[pl.ds(goff, _ZR)])
        return carry

    lax.fori_loop(0, _GARB // _ZR, zgarb, 0)

    def quad(p, carries):
        base = p * 2
        off = pl.multiple_of(base * 128, 256)
        pltpu.sync_copy(csrc_hbm.at[w, pl.ds(off, 256)], csrc_v)
        pltpu.sync_copy(cscat_hbm.at[w, pl.ds(off, 256)], cscat_v)
        handles = []
        for b in range(2):
            idx = csrc_v.at[pl.ds(b * 128, 128)]
            handles.append(pltpu.async_copy(m_hbm.at[idx], bufs[b], sems[b]))
        for b in range(2):
            t = base + b
            handles[b].wait()
            pltpu.sync_copy(same_hbm.at[w, pl.ds(pl.multiple_of(t * 2048, 2048), 2048)], same_v)
            rv = bufs[b]

            def rrow(j, cs, _rv=rv):
                sb = same_v[pl.ds(j * 16, 16)]
                new = []
                for k in range(8):
                    rr = _rv[j, pl.ds(k * 16, 16)]
                    ck = rr + sb * cs[k]
                    sv[j, pl.ds(k * 16, 16)] = ck
                    new.append(ck)
                return tuple(new)

            carries = lax.fori_loop(0, 128, rrow, carries)

            def cp(k2, cc, _b=b):
                dst_v[pl.ds(k2 * 16, 16)] = cscat_v[pl.ds(_b * 128 + k2 * 16, 16)]
                return cc

            lax.fori_loop(0, 8, cp, 0)
            pltpu.sync_copy(sv, acc.at[dst_v], add=True)
        return carries

    zc = tuple(jnp.zeros((16,), jnp.float32) for _ in range(8))
    lax.fori_loop(0, _NCHB // 2, quad, zc)

    out_off = pl.multiple_of(s * _RPT, _RPT)
    pltpu.sync_copy(acc.at[pl.ds(out_off, _RPT)],
                    out_hbm.at[pl.ds(pl.multiple_of(w * _RPT, _RPT), _RPT)])


@functools.cache
def _make_sc_calls():
    mesh = plsc.VectorSubcoreMesh(core_axis_name="c", subcore_axis_name="s")
    seg = pl.kernel(
        _seg_body,
        out_type=jax.ShapeDtypeStruct((_NROWS, _H), jnp.float32),
        mesh=mesh,
        scratch_types=[
            pltpu.VMEM((256,), jnp.int32),
            pltpu.VMEM((256,), jnp.int32),
            pltpu.VMEM((128,), jnp.int32),
            pltpu.VMEM((2048,), jnp.float32),
            pltpu.VMEM((128, _H), jnp.float32),
            pltpu.VMEM((128, _H), jnp.float32),
            pltpu.VMEM((128, _H), jnp.float32),
            pltpu.VMEM((_ZR, _H), jnp.float32),
            pltpu.VMEM_SHARED((_APAD, _H), jnp.float32),
            pltpu.SemaphoreType.DMA,
            pltpu.SemaphoreType.DMA,
        ],
    )
    return seg


def _edge_segment_sum(m, csrc, cscat, samebc):
    return _make_sc_calls()(m, csrc, cscat, samebc)


def _build_lists(src, dst):
    e = src.shape[0]
    order = jnp.argsort(dst, stable=True)
    srcs = src[order]
    dsts = dst[order]
    same = jnp.concatenate([jnp.zeros((1,), jnp.int32),
                            (dsts[1:] == dsts[:-1]).astype(jnp.int32)])
    last = jnp.concatenate([(dsts[1:] != dsts[:-1]).astype(jnp.int32),
                            jnp.ones((1,), jnp.int32)])
    owner = dsts // _RPT
    start = jnp.searchsorted(owner, jnp.arange(33, dtype=jnp.int32))
    idxmat = start[:32, None] + jnp.arange(_CAP, dtype=jnp.int32)[None, :]
    valid = idxmat < start[1:, None]
    cl = jnp.clip(idxmat, 0, e - 1)
    wvec = jnp.arange(32, dtype=jnp.int32)[:, None]
    jcyc = jnp.arange(_CAP, dtype=jnp.int32)[None, :] % _GARB
    garbl = _HALF + (wvec % 16) * _GARB + jcyc
    comp_src = jnp.where(valid, srcs[cl], 0)
    scat = jnp.where(last[cl] == 1, dsts[cl] - (wvec // 16) * _HALF, garbl)
    comp_scat = jnp.where(valid, scat, garbl)
    samef = jnp.where(valid, same[cl].astype(jnp.float32), 0.0)
    samebc = jnp.broadcast_to(samef[:, :, None], (32, _CAP, 16)).reshape(32, _CAP * 16)
    return comp_src, comp_scat, samebc


def kernel(x, edge_index, batch, num_graphs,
           W0, conv_weight, W_ih, W_hh, b_ih, b_hh, w1, b1, w2, b2):
    src = edge_index[0]
    dst = edge_index[1]

    bih2 = b_ih.reshape(1, 3 * _H)
    bhh2 = b_hh.reshape(1, 3 * _H)
    wms = jnp.concatenate([w1, w2], axis=1)
    bms = jnp.concatenate([b1, b2]).reshape(1, 2)
    batchf = batch.astype(jnp.float32).reshape(_N, 1)

    csrc, cscat, samebc = _build_lists(src, dst)

    x1, m = _init_call(x, W0, conv_weight[0])
    h = x1
    for i in range(2):
        agg = _edge_segment_sum(m, csrc, cscat, samebc)
        h, m = _gru_mid_call(agg, h, W_ih, W_hh, bih2, bhh2,
                             conv_weight[i + 1])
    agg = _edge_segment_sum(m, csrc, cscat, samebc)
    musig = _gru_head_call(agg, h, W_ih, W_hh, bih2, bhh2, wms, bms)
    sums = _sums_call(musig, batchf)
    mu_c = _apply_call(musig, batchf, sums)
    return (mu_c[:, 0], x1, musig[:, 1], musig[:, 0])


# speedup vs baseline: 1.8780x; 1.8285x over previous
import functools

import jax
import jax.numpy as jnp
from jax import lax
from jax.experimental import pallas as pl
from jax.experimental.pallas import tpu as pltpu
from jax.experimental.pallas import tpu_sc as plsc

_N = 10000
_D = 128
_H = 128
_G = 64
_RB = 400
_NRB = _N // _RB

_RPT = 320
_NROWS = 32 * _RPT
_HALF = 16 * _RPT
_GARB = 32
_APAD = _HALF + 16 * _GARB
_CAP = 10880
_NCHB = _CAP // 128
_ZR = 16


def _init_body(x_ref, w0_ref, wc_ref, x1_ref, m_ref):
    x1 = jax.nn.sigmoid(
        jnp.dot(x_ref[...], w0_ref[...], preferred_element_type=jnp.float32))
    x1_ref[...] = x1
    m_ref[...] = jnp.dot(x1, wc_ref[...], preferred_element_type=jnp.float32)


_init_call = pl.pallas_call(
    _init_body,
    grid=(_NRB,),
    in_specs=[
        pl.BlockSpec((_RB, _D), lambda i: (i, 0)),
        pl.BlockSpec((_D, _H), lambda i: (0, 0)),
        pl.BlockSpec((_H, _H), lambda i: (0, 0)),
    ],
    out_specs=[
        pl.BlockSpec((_RB, _H), lambda i: (i, 0)),
        pl.BlockSpec((_RB, _H), lambda i: (i, 0)),
    ],
    out_shape=[
        jax.ShapeDtypeStruct((_N, _H), jnp.float32),
        jax.ShapeDtypeStruct((_N, _H), jnp.float32),
    ],
)


def _gru_common(agg_ref, h_ref, wih_ref, whh_ref, bih_ref, bhh_ref):
    agg = agg_ref[...]
    h = h_ref[...]
    gi = jnp.dot(agg, wih_ref[...], preferred_element_type=jnp.float32) + bih_ref[...]
    gh = jnp.dot(h, whh_ref[...], preferred_element_type=jnp.float32) + bhh_ref[...]
    r = jax.nn.sigmoid(gi[:, :_H] + gh[:, :_H])
    z = jax.nn.sigmoid(gi[:, _H:2 * _H] + gh[:, _H:2 * _H])
    n = jnp.tanh(gi[:, 2 * _H:] + r * gh[:, 2 * _H:])
    return (1.0 - z) * n + z * h


def _gru_mid_body(agg_ref, h_ref, wih_ref, whh_ref, bih_ref, bhh_ref, wc_ref,
                  h_out_ref, m_out_ref):
    hn = _gru_common(agg_ref, h_ref, wih_ref, whh_ref, bih_ref, bhh_ref)
    h_out_ref[...] = hn
    m_out_ref[...] = jnp.dot(hn, wc_ref[...], preferred_element_type=jnp.float32)


_gru_mid_call = pl.pallas_call(
    _gru_mid_body,
    grid=(_NRB,),
    in_specs=[
        pl.BlockSpec((_RB, _H), lambda i: (i, 0)),
        pl.BlockSpec((_RB, _H), lambda i: (i, 0)),
        pl.BlockSpec((_H, 3 * _H), lambda i: (0, 0)),
        pl.BlockSpec((_H, 3 * _H), lambda i: (0, 0)),
        pl.BlockSpec((1, 3 * _H), lambda i: (0, 0)),
        pl.BlockSpec((1, 3 * _H), lambda i: (0, 0)),
        pl.BlockSpec((_H, _H), lambda i: (0, 0)),
    ],
    out_specs=[
        pl.BlockSpec((_RB, _H), lambda i: (i, 0)),
        pl.BlockSpec((_RB, _H), lambda i: (i, 0)),
    ],
    out_shape=[
        jax.ShapeDtypeStruct((_N, _H), jnp.float32),
        jax.ShapeDtypeStruct((_N, _H), jnp.float32),
    ],
)


def _gru_head_body(agg_ref, h_ref, wih_ref, whh_ref, bih_ref, bhh_ref,
                   wms_ref, bms_ref, ms_ref):
    hn = _gru_common(agg_ref, h_ref, wih_ref, whh_ref, bih_ref, bhh_ref)
    xo = jnp.maximum(hn, 0.0)
    ms = jnp.dot(xo, wms_ref[...], preferred_element_type=jnp.float32) + bms_ref[...]
    mu = ms[:, 0:1]
    s = ms[:, 1:2]
    sp = jnp.maximum(s, 0.0) + jnp.log1p(jnp.exp(-jnp.abs(s)))
    ms_ref[...] = jnp.concatenate([mu, sp], axis=1)


_gru_head_call = pl.pallas_call(
    _gru_head_body,
    grid=(_NRB,),
    in_specs=[
        pl.BlockSpec((_RB, _H), lambda i: (i, 0)),
        pl.BlockSpec((_RB, _H), lambda i: (i, 0)),
        pl.BlockSpec((_H, 3 * _H), lambda i: (0, 0)),
        pl.BlockSpec((_H, 3 * _H), lambda i: (0, 0)),
        pl.BlockSpec((1, 3 * _H), lambda i: (0, 0)),
        pl.BlockSpec((1, 3 * _H), lambda i: (0, 0)),
        pl.BlockSpec((_H, 2), lambda i: (0, 0)),
        pl.BlockSpec((1, 2), lambda i: (0, 0)),
    ],
    out_specs=pl.BlockSpec((_RB, 2), lambda i: (i, 0)),
    out_shape=jax.ShapeDtypeStruct((_N, 2), jnp.float32),
)


def _sums_body(ms_ref, b_ref, out_ref):
    i = pl.program_id(0)

    @pl.when(i == 0)
    def _():
        out_ref[...] = jnp.zeros_like(out_ref)

    b = b_ref[:, 0]
    onehot = (b[None, :] == lax.broadcasted_iota(jnp.int32, (_G, _RB), 0).astype(jnp.float32))
    out_ref[...] += jnp.dot(onehot.astype(jnp.float32), ms_ref[...],
                            preferred_element_type=jnp.float32,
                            precision=lax.Precision.HIGHEST)


_sums_call = pl.pallas_call(
    _sums_body,
    grid=(_NRB,),
    in_specs=[
        pl.BlockSpec((_RB, 2), lambda i: (i, 0)),
        pl.BlockSpec((_RB, 1), lambda i: (i, 0)),
    ],
    out_specs=pl.BlockSpec((_G, 2), lambda i: (0, 0)),
    out_shape=jax.ShapeDtypeStruct((_G, 2), jnp.float32),
)


def _apply_body(ms_ref, b_ref, sums_ref, out_ref):
    b = b_ref[:, 0]
    onehot = (b[:, None] == lax.broadcasted_iota(jnp.int32, (_RB, _G), 1).astype(jnp.float32))
    gath = jnp.dot(onehot.astype(jnp.float32), sums_ref[...],
                   preferred_element_type=jnp.float32,
                   precision=lax.Precision.HIGHEST)
    mu = ms_ref[:, 0:1]
    sig = ms_ref[:, 1:2]
    out_ref[...] = mu - gath[:, 0:1] * (sig / gath[:, 1:2])


_apply_call = pl.pallas_call(
    _apply_body,
    grid=(_NRB,),
    in_specs=[
        pl.BlockSpec((_RB, 2), lambda i: (i, 0)),
        pl.BlockSpec((_RB, 1), lambda i: (i, 0)),
        pl.BlockSpec((_G, 2), lambda i: (0, 0)),
    ],
    out_specs=pl.BlockSpec((_RB, 1), lambda i: (i, 0)),
    out_shape=jax.ShapeDtypeStruct((_N, 1), jnp.float32),
)


def _seg_body(m_hbm, csrc_hbm, cscat_hbm, same_hbm, out_hbm,
              csrc_v, cscat_v, dst_v, same_v, rv, sv, zbuf, acc, sem):
    c = lax.axis_index("c")
    s = lax.axis_index("s")
    w = c * 16 + s

    pltpu.sync_copy(csrc_hbm.at[w], csrc_v)
    pltpu.sync_copy(cscat_hbm.at[w], cscat_v)

    zero16 = jnp.zeros((16,), jnp.float32)

    def zrow(k, carry):
        zbuf[k // 8, pl.ds((k % 8) * 16, 16)] = zero16
        return carry

    lax.fori_loop(0, _ZR * 8, zrow, 0)

    def zslice(j, carry):
        off = pl.multiple_of(s * _RPT + j * _ZR, _ZR)
        pltpu.sync_copy(zbuf, acc.at[pl.ds(off, _ZR)])
        return carry

    lax.fori_loop(0, _RPT // _ZR, zslice, 0)

    def zgarb(j, carry):
        goff = pl.multiple_of(_HALF + s * _GARB + j * _ZR, _ZR)
        pltpu.sync_copy(zbuf, acc.at[pl.ds(goff, _ZR)])
        return carry

    lax.fori_loop(0, _GARB // _ZR, zgarb, 0)

    def ebody(t, carries):
        idx = csrc_v.at[pl.ds(t * 128, 128)]
        pltpu.async_copy(m_hbm.at[idx], rv, sem).wait()
        pltpu.sync_copy(same_hbm.at[w, pl.ds(pl.multiple_of(t * 2048, 2048), 2048)],
                        same_v)

        def rrow(j, cs):
            sb = same_v[pl.ds(j * 16, 16)]
            new = []
            for k in range(8):
                rr = rv[j, pl.ds(k * 16, 16)]
                ck = rr + sb * cs[k]
                sv[j, pl.ds(k * 16, 16)] = ck
                new.append(ck)
            return tuple(new)

        carries = lax.fori_loop(0, 128, rrow, carries)

        def cp(k2, cc):
            dst_v[pl.ds(k2 * 16, 16)] = cscat_v[pl.ds(t * 128 + k2 * 16, 16)]
            return cc

        lax.fori_loop(0, 8, cp, 0)
        pltpu.sync_copy(sv, acc.at[dst_v], add=True)
        return carries

    zc = tuple(jnp.zeros((16,), jnp.float32) for _ in range(8))
    lax.fori_loop(0, _NCHB, ebody, zc)

    out_off = pl.multiple_of(s * _RPT, _RPT)
    pltpu.sync_copy(acc.at[pl.ds(out_off, _RPT)],
                    out_hbm.at[pl.ds(pl.multiple_of(w * _RPT, _RPT), _RPT)])


@functools.cache
def _make_sc_calls():
    mesh = plsc.VectorSubcoreMesh(core_axis_name="c", subcore_axis_name="s")
    seg = pl.kernel(
        _seg_body,
        out_type=jax.ShapeDtypeStruct((_NROWS, _H), jnp.float32),
        mesh=mesh,
        scratch_types=[
            pltpu.VMEM((_CAP,), jnp.int32),
            pltpu.VMEM((_CAP,), jnp.int32),
            pltpu.VMEM((128,), jnp.int32),
            pltpu.VMEM((2048,), jnp.float32),
            pltpu.VMEM((128, _H), jnp.float32),
            pltpu.VMEM((128, _H), jnp.float32),
            pltpu.VMEM((_ZR, _H), jnp.float32),
            pltpu.VMEM_SHARED((_APAD, _H), jnp.float32),
            pltpu.SemaphoreType.DMA,
        ],
    )
    return seg


def _edge_segment_sum(m, csrc, cscat, samebc):
    return _make_sc_calls()(m, csrc, cscat, samebc)


def _build_lists(src, dst):
    e = src.shape[0]
    order = jnp.argsort(dst, stable=True)
    srcs = src[order]
    dsts = dst[order]
    same = jnp.concatenate([jnp.zeros((1,), jnp.int32),
                            (dsts[1:] == dsts[:-1]).astype(jnp.int32)])
    last = jnp.concatenate([(dsts[1:] != dsts[:-1]).astype(jnp.int32),
                            jnp.ones((1,), jnp.int32)])
    owner = dsts // _RPT
    start = jnp.searchsorted(owner, jnp.arange(33, dtype=jnp.int32))
    idxmat = start[:32, None] + jnp.arange(_CAP, dtype=jnp.int32)[None, :]
    valid = idxmat < start[1:, None]
    cl = jnp.clip(idxmat, 0, e - 1)
    wvec = jnp.arange(32, dtype=jnp.int32)[:, None]
    jcyc = jnp.arange(_CAP, dtype=jnp.int32)[None, :] % _GARB
    garbl = _HALF + (wvec % 16) * _GARB + jcyc
    comp_src = jnp.where(valid, srcs[cl], 0)
    scat = jnp.where(last[cl] == 1, dsts[cl] - (wvec // 16) * _HALF, garbl)
    comp_scat = jnp.where(valid, scat, garbl)
    samef = jnp.where(valid, same[cl].astype(jnp.float32), 0.0)
    samebc = jnp.broadcast_to(samef[:, :, None], (32, _CAP, 16)).reshape(32, _CAP * 16)
    return comp_src, comp_scat, samebc


def kernel(x, edge_index, batch, num_graphs,
           W0, conv_weight, W_ih, W_hh, b_ih, b_hh, w1, b1, w2, b2):
    src = edge_index[0]
    dst = edge_index[1]

    bih2 = b_ih.reshape(1, 3 * _H)
    bhh2 = b_hh.reshape(1, 3 * _H)
    wms = jnp.concatenate([w1, w2], axis=1)
    bms = jnp.concatenate([b1, b2]).reshape(1, 2)
    batchf = batch.astype(jnp.float32).reshape(_N, 1)

    csrc, cscat, samebc = _build_lists(src, dst)

    x1, m = _init_call(x, W0, conv_weight[0])
    h = x1
    for i in range(2):
        agg = _edge_segment_sum(m, csrc, cscat, samebc)
        h, m = _gru_mid_call(agg, h, W_ih, W_hh, bih2, bhh2,
                             conv_weight[i + 1])
    agg = _edge_segment_sum(m, csrc, cscat, samebc)
    musig = _gru_head_call(agg, h, W_ih, W_hh, bih2, bhh2, wms, bms)
    sums = _sums_call(musig, batchf)
    mu_c = _apply_call(musig, batchf, sums)
    return (mu_c[:, 0], x1, musig[:, 1], musig[:, 0])
